# Initial kernel scaffold; baseline (speedup 1.0000x reference)
import jax
import jax.numpy as jnp
from jax.experimental import pallas as pl


def kernel(xyz, feature, points, W0, b0, g0, beta0, W1, b1, g1, beta1, W2, b2, g2, beta2):
    raise NotImplementedError("write your pallas kernel here")


# speedup vs baseline: 4.1184x; 4.1184x over previous
import functools

import jax
import jax.numpy as jnp
from jax import lax
from jax.experimental import pallas as pl
from jax.experimental.pallas import tpu as pltpu
from jax.experimental.pallas import tpu_sc as plsc

B, N, S, K = 8, 2048, 512, 32
D_F, D_P = 20, 64
C0 = 24
CT = 128
ROWS = B * S * K
RT = 4096
NSTEP = ROWS // RT

_f32 = jnp.float32
_i32 = jnp.int32


def _fps_body(c_ref, out_ref):
    cb = c_ref[0]
    lane_s = lax.broadcasted_iota(_i32, (S,), 0)

    def body(i, state):
        dist, far, cent = state
        cent = jnp.where(lane_s == i, far, cent)
        c = c_ref[0, pl.ds(far, 1), :]
        d = jnp.sum((cb - c) ** 2, axis=1)
        dist = jnp.minimum(dist, d)
        far = jnp.argmax(dist).astype(_i32)
        return dist, far, cent

    init = (jnp.full((N,), 1e10, _f32), jnp.int32(0), jnp.zeros((S,), _i32))
    _, _, cent = lax.fori_loop(0, S, body, init)
    out_ref[0, 0, :] = cent


def _run_fps(combined):
    return pl.pallas_call(
        _fps_body,
        grid=(B,),
        in_specs=[pl.BlockSpec((1, N, C0), lambda b: (b, 0, 0))],
        out_specs=pl.BlockSpec((1, 1, S), lambda b: (b, 0, 0)),
        out_shape=jax.ShapeDtypeStruct((B, 1, S), _i32),
    )(combined)


def _knn_body(c_ref, fps_ref, q_ref, idx_ref):
    cb = c_ref[0]
    fps = fps_ref[0, 0, :]
    onehot = (fps[:, None] ==
              lax.broadcasted_iota(_i32, (S, N), 1)).astype(_f32)
    q = jax.lax.dot_general(onehot, cb, (((1,), (0,)), ((), ())),
                            preferred_element_type=_f32)
    q_ref[0] = q
    nq = jnp.sum(q * q, axis=1)
    nx = jnp.sum(cb * cb, axis=1)
    dist = (-2.0 * jax.lax.dot_general(q, cb, (((1,), (1,)), ((), ())),
                                       preferred_element_type=_f32)
            + nq[:, None] + nx[None, :])
    off = pl.program_id(0) * N
    lane_n = lax.broadcasted_iota(_i32, (S, N), 1)
    cols = []
    for _ in range(K):
        m = jnp.argmin(dist, axis=1).astype(_i32)
        cols.append(m + off)
        dist = jnp.where(lane_n == m[:, None], 1e30, dist)
    idx_ref[0] = jnp.stack(cols, axis=1)


def _run_knn(combined, fps):
    return pl.pallas_call(
        _knn_body,
        grid=(B,),
        in_specs=[
            pl.BlockSpec((1, N, C0), lambda b: (b, 0, 0)),
            pl.BlockSpec((1, 1, S), lambda b: (b, 0, 0)),
        ],
        out_specs=[
            pl.BlockSpec((1, S, C0), lambda b: (b, 0, 0)),
            pl.BlockSpec((1, S, K), lambda b: (b, 0, 0)),
        ],
        out_shape=[
            jax.ShapeDtypeStruct((B, S, C0), _f32),
            jax.ShapeDtypeStruct((B, S, K), _i32),
        ],
    )(combined, fps)


def _sc_gather(table, idx):
    info = plsc.get_sparse_core_info()
    nw = info.num_cores * info.num_subcores
    b_per_w = ROWS // nw
    chunk = 512
    nchunks = b_per_w // chunk
    mesh = plsc.VectorSubcoreMesh(core_axis_name="c", subcore_axis_name="s")

    @functools.partial(
        pl.kernel,
        mesh=mesh,
        out_type=jax.ShapeDtypeStruct((ROWS, CT), _f32),
        scratch_types=[
            pltpu.VMEM((chunk,), _i32),
            pltpu.VMEM((chunk, CT), _f32),
            pltpu.SemaphoreType.DMA,
        ],
    )
    def gather_k(table_hbm, idx_hbm, out_hbm, idx_v, rows_v, sem):
        wid = lax.axis_index("s") * info.num_cores + lax.axis_index("c")
        base = wid * b_per_w

        def body(ci, carry):
            off = base + ci * chunk
            pltpu.sync_copy(idx_hbm.at[pl.ds(off, chunk)], idx_v)
            pltpu.async_copy(table_hbm.at[idx_v], rows_v, sem).wait()
            pltpu.sync_copy(rows_v, out_hbm.at[pl.ds(off, chunk)])
            return carry

        lax.fori_loop(0, nchunks, body, 0)

    return gather_k(table, idx)


def _l0_body(g_ref, qc_ref, wg_ref, wq_ref, p_ref, y_ref, s0_ref, s2_ref):
    g = g_ref[...]
    qc = qc_ref[...]
    y = jax.lax.dot_general(g, wg_ref[...], (((1,), (1,)), ((), ())),
                            preferred_element_type=_f32)
    y = y + jax.lax.dot_general(qc, wq_ref[...], (((1,), (1,)), ((), ())),
                                preferred_element_type=_f32)
    gx = g[:, 0:3] - qc[:, 0:3]
    sq = jnp.sum(gx * gx, axis=1)
    nrm = jnp.where(sq > 0, jnp.sqrt(jnp.where(sq > 0, sq, 1.0)), 0.0)
    y = y + nrm[:, None] * p_ref[0:1, :] + p_ref[1:2, :]
    y_ref[...] = y

    @pl.when(pl.program_id(0) == 0)
    def _():
        s0_ref[...] = jnp.zeros_like(s0_ref)
        s2_ref[...] = jnp.zeros_like(s2_ref)

    s0_ref[...] += jnp.sum(y, axis=0)[None, None, :]
    s2_ref[...] += jnp.sum(y * y, axis=0)[None, None, :]


def _run_l0(g, qc, wg, wq, p0):
    return pl.pallas_call(
        _l0_body,
        grid=(NSTEP,),
        in_specs=[
            pl.BlockSpec((RT, CT), lambda i: (i, 0)),
            pl.BlockSpec((RT, C0), lambda i: (i, 0)),
            pl.BlockSpec((64, CT), lambda i: (0, 0)),
            pl.BlockSpec((64, C0), lambda i: (0, 0)),
            pl.BlockSpec((8, 64), lambda i: (0, 0)),
        ],
        out_specs=[
            pl.BlockSpec((RT, 64), lambda i: (i, 0)),
            pl.BlockSpec((1, 1, 64), lambda i: (0, 0, 0)),
            pl.BlockSpec((1, 1, 64), lambda i: (0, 0, 0)),
        ],
        out_shape=[
            jax.ShapeDtypeStruct((ROWS, 64), _f32),
            jax.ShapeDtypeStruct((1, 1, 64), _f32),
            jax.ShapeDtypeStruct((1, 1, 64), _f32),
        ],
    )(g, qc, wg, wq, p0)


def _mid_body(y_ref, w_ref, p_ref, o_ref, s0_ref, s2_ref):
    z = jnp.maximum(y_ref[...] * p_ref[0:1, 0:64] + p_ref[1:2, 0:64], 0.0)
    y = jax.lax.dot_general(z, w_ref[...], (((1,), (1,)), ((), ())),
                            preferred_element_type=_f32)
    y = y + p_ref[2:3, 0:w_ref.shape[0]]
    o_ref[...] = y

    @pl.when(pl.program_id(0) == 0)
    def _():
        s0_ref[...] = jnp.zeros_like(s0_ref)
        s2_ref[...] = jnp.zeros_like(s2_ref)

    s0_ref[...] += jnp.sum(y, axis=0)[None, None, :]
    s2_ref[...] += jnp.sum(y * y, axis=0)[None, None, :]


def _run_mid(y, w, p, out_ch):
    return pl.pallas_call(
        _mid_body,
        grid=(NSTEP,),
        in_specs=[
            pl.BlockSpec((RT, 64), lambda i: (i, 0)),
            pl.BlockSpec((out_ch, 64), lambda i: (0, 0)),
            pl.BlockSpec((8, 128), lambda i: (0, 0)),
        ],
        out_specs=[
            pl.BlockSpec((RT, out_ch), lambda i: (i, 0)),
            pl.BlockSpec((1, 1, out_ch), lambda i: (0, 0, 0)),
            pl.BlockSpec((1, 1, out_ch), lambda i: (0, 0, 0)),
        ],
        out_shape=[
            jax.ShapeDtypeStruct((ROWS, out_ch), _f32),
            jax.ShapeDtypeStruct((1, 1, out_ch), _f32),
            jax.ShapeDtypeStruct((1, 1, out_ch), _f32),
        ],
    )(y, w, p)


def _pool_body(y_ref, p_ref, o_ref):
    a = p_ref[0:1, :].reshape(1, 1, 128)
    c = p_ref[1:2, :].reshape(1, 1, 128)
    z = jnp.maximum(y_ref[...] * a + c, 0.0)
    o_ref[...] = jnp.max(z, axis=1)


def _run_pool(y3, p):
    return pl.pallas_call(
        _pool_body,
        grid=(NSTEP,),
        in_specs=[
            pl.BlockSpec((RT // K, K, 128), lambda i: (i, 0, 0)),
            pl.BlockSpec((8, 128), lambda i: (0, 0)),
        ],
        out_specs=pl.BlockSpec((RT // K, 128), lambda i: (i, 0)),
        out_shape=jax.ShapeDtypeStruct((B * S, 128), _f32),
    )(y3, p)


def _bn_affine(s0, s2, g, beta):
    mean = s0.reshape(-1) / ROWS
    var = s2.reshape(-1) / ROWS - mean * mean
    a = g / jnp.sqrt(var + 1e-5)
    return a, beta - mean * a


def kernel(xyz, feature, points, W0, b0, g0, beta0, W1, b1, g1, beta1,
           W2, b2, g2, beta2):
    xyz_t = jnp.transpose(xyz, (0, 2, 1))
    pts_t = jnp.transpose(points, (0, 2, 1))
    combined = jnp.concatenate(
        [xyz_t, feature, jnp.zeros((B, N, 1), _f32)], axis=-1)

    fps = _run_fps(combined)
    qfull, gidx = _run_knn(combined, fps)

    table = jnp.concatenate(
        [xyz_t, feature, pts_t, jnp.zeros((B, N, CT - 87), _f32)],
        axis=-1).reshape(B * N, CT)
    g = _sc_gather(table, gidx.reshape(ROWS))

    qc = jnp.repeat(qfull.reshape(B * S, C0), K, axis=0)

    wg = jnp.concatenate(
        [W0[:, 0:3], W0[:, 4:24], W0[:, 24:88],
         jnp.zeros((64, CT - 87), _f32)], axis=1)
    wq = jnp.concatenate(
        [-W0[:, 0:3], -W0[:, 4:24], jnp.zeros((64, 1), _f32)],
        axis=1)
    p0 = jnp.zeros((8, 64), _f32).at[0].set(W0[:, 3]).at[1].set(b0)

    y0, s0a, s0b = _run_l0(g, qc, wg, wq, p0)
    a0, c0 = _bn_affine(s0a, s0b, g0, beta0)

    p1 = (jnp.zeros((8, 128), _f32)
          .at[0, 0:64].set(a0).at[1, 0:64].set(c0).at[2, 0:64].set(b1))
    y1, s1a, s1b = _run_mid(y0, W1, p1, 64)
    a1, c1 = _bn_affine(s1a, s1b, g1, beta1)

    p2 = (jnp.zeros((8, 128), _f32)
          .at[0, 0:64].set(a1).at[1, 0:64].set(c1).at[2, 0:128].set(b2))
    y2, s2a, s2b = _run_mid(y1, W2, p2, 128)
    a2, c2 = _bn_affine(s2a, s2b, g2, beta2)

    p3 = jnp.zeros((8, 128), _f32).at[0].set(a2).at[1].set(c2)
    pooled = _run_pool(y2.reshape(B * S, K, 128), p3)

    new_xyz = jnp.transpose(qfull[:, :, 0:3], (0, 2, 1))
    new_bin_map = qfull[:, :, 3:23]
    new_points_out = jnp.transpose(pooled.reshape(B, S, 128), (0, 2, 1))
    return new_xyz, new_points_out, new_bin_map


# speedup vs baseline: 7.6655x; 1.8613x over previous
import functools

import jax
import jax.numpy as jnp
from jax import lax
from jax.experimental import pallas as pl
from jax.experimental.pallas import tpu as pltpu
from jax.experimental.pallas import tpu_sc as plsc

B, N, S, K = 8, 2048, 512, 32
D_F, D_P = 20, 64
C0 = 24
CT = 128
ROWS = B * S * K
RT = 4096
NSTEP = ROWS // RT

_f32 = jnp.float32
_i32 = jnp.int32


def _fps_body(c_ref, out_ref):
    cb = c_ref[0]
    lane_s = lax.broadcasted_iota(_i32, (S,), 0)

    def body(i, state):
        dist, far, cent = state
        cent = jnp.where(lane_s == i, far, cent)
        c = c_ref[0, pl.ds(far, 1), :]
        d = jnp.sum((cb - c) ** 2, axis=1)
        dist = jnp.minimum(dist, d)
        far = jnp.argmax(dist).astype(_i32)
        return dist, far, cent

    init = (jnp.full((N,), 1e10, _f32), jnp.int32(0), jnp.zeros((S,), _i32))
    _, _, cent = lax.fori_loop(0, S, body, init)
    out_ref[0, 0, :] = cent


def _run_fps(combined):
    return pl.pallas_call(
        _fps_body,
        grid=(B,),
        in_specs=[pl.BlockSpec((1, N, C0), lambda b: (b, 0, 0))],
        out_specs=pl.BlockSpec((1, 1, S), lambda b: (b, 0, 0)),
        out_shape=jax.ShapeDtypeStruct((B, 1, S), _i32),
    )(combined)


def _fps_body_t(t_ref, out_ref):
    xt = t_ref[...]
    iota_row = lax.broadcasted_iota(_i32, (1, N), 1)
    lane_row = lax.broadcasted_iota(_i32, (1, S), 1)

    def body(i, state):
        dist, idxs, cents = state
        cents = [jnp.where(lane_row == i, idxs[b], cents[b])
                 for b in range(B)]
        masks = [(iota_row == idxs[b]).astype(_f32) for b in range(B)]
        cs = [jax.lax.dot_general(
                  xt[b], masks[b], (((1,), (1,)), ((), ())),
                  preferred_element_type=_f32)[None]
              for b in range(B)]
        c = jnp.concatenate(cs, axis=0)
        sq = xt - c
        d = jnp.sum(sq * sq, axis=1)
        dist = jnp.minimum(dist, d)
        idxs = []
        for b in range(B):
            row = dist[b]
            m = jnp.max(row)
            idxs.append(jnp.min(jnp.where(row == m, iota_row[0],
                                          jnp.int32(N))))
        return dist, idxs, cents

    init = (jnp.full((B, N), 1e10, _f32), [jnp.int32(0)] * B,
            [jnp.zeros((1, S), _i32)] * B)
    _, _, cents = lax.fori_loop(0, S, body, init)
    out_ref[...] = jnp.concatenate(cents, axis=0)[:, None, :]


def _run_fps_t(combined_t):
    return pl.pallas_call(
        _fps_body_t,
        grid=(1,),
        in_specs=[pl.BlockSpec((B, C0, N), lambda i: (0, 0, 0))],
        out_specs=pl.BlockSpec((B, 1, S), lambda i: (0, 0, 0)),
        out_shape=jax.ShapeDtypeStruct((B, 1, S), _i32),
    )(combined_t)


def _knn_body(c_ref, fps_ref, q_ref, idx_ref):
    cb = c_ref[0]
    fps = fps_ref[0, 0, :]
    onehot = (fps[:, None] ==
              lax.broadcasted_iota(_i32, (S, N), 1)).astype(_f32)
    q = jax.lax.dot_general(onehot, cb, (((1,), (0,)), ((), ())),
                            preferred_element_type=_f32)
    q_ref[0] = q
    qf, qx = q[:, 3:23], q[:, 0:3]
    cf, cx = cb[:, 3:23], cb[:, 0:3]
    d1 = -2.0 * jax.lax.dot_general(qf, cf, (((1,), (1,)), ((), ())),
                                    preferred_element_type=_f32)
    d1 = d1 + jnp.sum(qf * qf, axis=1)[:, None]
    d1 = d1 + jnp.sum(cf * cf, axis=1)[None, :]
    d2 = -2.0 * jax.lax.dot_general(qx, cx, (((1,), (1,)), ((), ())),
                                    preferred_element_type=_f32)
    d2 = d2 + jnp.sum(qx * qx, axis=1)[:, None]
    d2 = d2 + jnp.sum(cx * cx, axis=1)[None, :]
    dist = d1 + d2
    off = pl.program_id(0) * N
    lane_n = lax.broadcasted_iota(_i32, (S, N), 1)
    cols = []
    for _ in range(K):
        m = jnp.argmin(dist, axis=1).astype(_i32)
        cols.append(m + off)
        dist = jnp.where(lane_n == m[:, None], 1e30, dist)
    idx_ref[0] = jnp.stack(cols, axis=1)


def _run_knn(combined, fps):
    return pl.pallas_call(
        _knn_body,
        grid=(B,),
        in_specs=[
            pl.BlockSpec((1, N, C0), lambda b: (b, 0, 0)),
            pl.BlockSpec((1, 1, S), lambda b: (b, 0, 0)),
        ],
        out_specs=[
            pl.BlockSpec((1, S, C0), lambda b: (b, 0, 0)),
            pl.BlockSpec((1, S, K), lambda b: (b, 0, 0)),
        ],
        out_shape=[
            jax.ShapeDtypeStruct((B, S, C0), _f32),
            jax.ShapeDtypeStruct((B, S, K), _i32),
        ],
    )(combined, fps)


def _sc_gather(table, idx):
    info = plsc.get_sparse_core_info()
    nw = info.num_cores * info.num_subcores
    b_per_w = ROWS // nw
    chunk = 512
    nchunks = b_per_w // chunk
    mesh = plsc.VectorSubcoreMesh(core_axis_name="c", subcore_axis_name="s")

    @functools.partial(
        pl.kernel,
        mesh=mesh,
        out_type=jax.ShapeDtypeStruct((ROWS, CT), _f32),
        scratch_types=[
            pltpu.VMEM((chunk,), _i32),
            pltpu.VMEM((chunk, CT), _f32),
            pltpu.SemaphoreType.DMA,
        ],
    )
    def gather_k(table_hbm, idx_hbm, out_hbm, idx_v, rows_v, sem):
        wid = lax.axis_index("s") * info.num_cores + lax.axis_index("c")
        base = wid * b_per_w

        def body(ci, carry):
            off = base + ci * chunk
            pltpu.sync_copy(idx_hbm.at[pl.ds(off, chunk)], idx_v)
            pltpu.async_copy(table_hbm.at[idx_v], rows_v, sem).wait()
            pltpu.sync_copy(rows_v, out_hbm.at[pl.ds(off, chunk)])
            return carry

        lax.fori_loop(0, nchunks, body, 0)

    return gather_k(table, idx)


def _l0_body(g_ref, qc_ref, wg_ref, wq_ref, p_ref, y_ref, s0_ref, s2_ref):
    g = g_ref[...]
    qc = qc_ref[...]
    y = jax.lax.dot_general(g, wg_ref[...], (((1,), (1,)), ((), ())),
                            preferred_element_type=_f32)
    y = y + jax.lax.dot_general(qc, wq_ref[...], (((1,), (1,)), ((), ())),
                                preferred_element_type=_f32)
    gx = g[:, 0:3] - qc[:, 0:3]
    sq = jnp.sum(gx * gx, axis=1)
    nrm = jnp.where(sq > 0, jnp.sqrt(jnp.where(sq > 0, sq, 1.0)), 0.0)
    y = y + nrm[:, None] * p_ref[0:1, :] + p_ref[1:2, :]
    y_ref[...] = y

    @pl.when(pl.program_id(0) == 0)
    def _():
        s0_ref[...] = jnp.zeros_like(s0_ref)
        s2_ref[...] = jnp.zeros_like(s2_ref)

    s0_ref[...] += jnp.sum(y, axis=0)[None, None, :]
    s2_ref[...] += jnp.sum(y * y, axis=0)[None, None, :]


def _run_l0(g, qc, wg, wq, p0):
    return pl.pallas_call(
        _l0_body,
        grid=(NSTEP,),
        in_specs=[
            pl.BlockSpec((RT, CT), lambda i: (i, 0)),
            pl.BlockSpec((RT, C0), lambda i: (i, 0)),
            pl.BlockSpec((64, CT), lambda i: (0, 0)),
            pl.BlockSpec((64, C0), lambda i: (0, 0)),
            pl.BlockSpec((8, 64), lambda i: (0, 0)),
        ],
        out_specs=[
            pl.BlockSpec((RT, 64), lambda i: (i, 0)),
            pl.BlockSpec((1, 1, 64), lambda i: (0, 0, 0)),
            pl.BlockSpec((1, 1, 64), lambda i: (0, 0, 0)),
        ],
        out_shape=[
            jax.ShapeDtypeStruct((ROWS, 64), _f32),
            jax.ShapeDtypeStruct((1, 1, 64), _f32),
            jax.ShapeDtypeStruct((1, 1, 64), _f32),
        ],
    )(g, qc, wg, wq, p0)


def _mid_body(y_ref, w_ref, p_ref, o_ref, s0_ref, s2_ref):
    z = jnp.maximum(y_ref[...] * p_ref[0:1, 0:64] + p_ref[1:2, 0:64], 0.0)
    y = jax.lax.dot_general(z, w_ref[...], (((1,), (1,)), ((), ())),
                            preferred_element_type=_f32)
    y = y + p_ref[2:3, 0:w_ref.shape[0]]
    o_ref[...] = y

    @pl.when(pl.program_id(0) == 0)
    def _():
        s0_ref[...] = jnp.zeros_like(s0_ref)
        s2_ref[...] = jnp.zeros_like(s2_ref)

    s0_ref[...] += jnp.sum(y, axis=0)[None, None, :]
    s2_ref[...] += jnp.sum(y * y, axis=0)[None, None, :]


def _run_mid(y, w, p, out_ch):
    return pl.pallas_call(
        _mid_body,
        grid=(NSTEP,),
        in_specs=[
            pl.BlockSpec((RT, 64), lambda i: (i, 0)),
            pl.BlockSpec((out_ch, 64), lambda i: (0, 0)),
            pl.BlockSpec((8, 128), lambda i: (0, 0)),
        ],
        out_specs=[
            pl.BlockSpec((RT, out_ch), lambda i: (i, 0)),
            pl.BlockSpec((1, 1, out_ch), lambda i: (0, 0, 0)),
            pl.BlockSpec((1, 1, out_ch), lambda i: (0, 0, 0)),
        ],
        out_shape=[
            jax.ShapeDtypeStruct((ROWS, out_ch), _f32),
            jax.ShapeDtypeStruct((1, 1, out_ch), _f32),
            jax.ShapeDtypeStruct((1, 1, out_ch), _f32),
        ],
    )(y, w, p)


def _pool_body(y_ref, p_ref, o_ref):
    a = p_ref[0:1, :].reshape(1, 1, 128)
    c = p_ref[1:2, :].reshape(1, 1, 128)
    z = jnp.maximum(y_ref[...] * a + c, 0.0)
    o_ref[...] = jnp.max(z, axis=1)


def _run_pool(y3, p):
    return pl.pallas_call(
        _pool_body,
        grid=(NSTEP,),
        in_specs=[
            pl.BlockSpec((RT // K, K, 128), lambda i: (i, 0, 0)),
            pl.BlockSpec((8, 128), lambda i: (0, 0)),
        ],
        out_specs=pl.BlockSpec((RT // K, 128), lambda i: (i, 0)),
        out_shape=jax.ShapeDtypeStruct((B * S, 128), _f32),
    )(y3, p)


def _bn_affine(s0, s2, g, beta):
    mean = s0.reshape(-1) / ROWS
    var = s2.reshape(-1) / ROWS - mean * mean
    a = g / jnp.sqrt(var + 1e-5)
    return a, beta - mean * a


def kernel(xyz, feature, points, W0, b0, g0, beta0, W1, b1, g1, beta1,
           W2, b2, g2, beta2):
    xyz_t = jnp.transpose(xyz, (0, 2, 1))
    pts_t = jnp.transpose(points, (0, 2, 1))
    combined = jnp.concatenate(
        [xyz_t, feature, jnp.zeros((B, N, 1), _f32)], axis=-1)

    fps = _run_fps_t(jnp.transpose(combined, (0, 2, 1)))
    qfull, gidx = _run_knn(combined, fps)

    table = jnp.concatenate(
        [xyz_t, feature, pts_t, jnp.zeros((B, N, CT - 87), _f32)],
        axis=-1).reshape(B * N, CT)
    g = _sc_gather(table, gidx.reshape(ROWS))

    qc = jnp.repeat(qfull.reshape(B * S, C0), K, axis=0)

    wg = jnp.concatenate(
        [W0[:, 0:3], W0[:, 4:24], W0[:, 24:88],
         jnp.zeros((64, CT - 87), _f32)], axis=1)
    wq = jnp.concatenate(
        [-W0[:, 0:3], -W0[:, 4:24], jnp.zeros((64, 1), _f32)],
        axis=1)
    p0 = jnp.zeros((8, 64), _f32).at[0].set(W0[:, 3]).at[1].set(b0)

    y0, s0a, s0b = _run_l0(g, qc, wg, wq, p0)
    a0, c0 = _bn_affine(s0a, s0b, g0, beta0)

    p1 = (jnp.zeros((8, 128), _f32)
          .at[0, 0:64].set(a0).at[1, 0:64].set(c0).at[2, 0:64].set(b1))
    y1, s1a, s1b = _run_mid(y0, W1, p1, 64)
    a1, c1 = _bn_affine(s1a, s1b, g1, beta1)

    p2 = (jnp.zeros((8, 128), _f32)
          .at[0, 0:64].set(a1).at[1, 0:64].set(c1).at[2, 0:128].set(b2))
    y2, s2a, s2b = _run_mid(y1, W2, p2, 128)
    a2, c2 = _bn_affine(s2a, s2b, g2, beta2)

    p3 = jnp.zeros((8, 128), _f32).at[0].set(a2).at[1].set(c2)
    pooled = _run_pool(y2.reshape(B * S, K, 128), p3)

    new_xyz = jnp.transpose(qfull[:, :, 0:3], (0, 2, 1))
    new_bin_map = qfull[:, :, 3:23]
    new_points_out = jnp.transpose(pooled.reshape(B, S, 128), (0, 2, 1))
    return new_xyz, new_points_out, new_bin_map


# speedup vs baseline: 7.8434x; 1.0232x over previous
import functools

import jax
import jax.numpy as jnp
from jax import lax
from jax.experimental import pallas as pl
from jax.experimental.pallas import tpu as pltpu
from jax.experimental.pallas import tpu_sc as plsc

B, N, S, K = 8, 2048, 512, 32
D_F, D_P = 20, 64
C0 = 24
CT = 128
ROWS = B * S * K
RT = 4096
NSTEP = ROWS // RT

_f32 = jnp.float32
_i32 = jnp.int32


def _fps_body(c_ref, out_ref):
    cb = c_ref[0]
    lane_s = lax.broadcasted_iota(_i32, (S,), 0)

    def body(i, state):
        dist, far, cent = state
        cent = jnp.where(lane_s == i, far, cent)
        c = c_ref[0, pl.ds(far, 1), :]
        d = jnp.sum((cb - c) ** 2, axis=1)
        dist = jnp.minimum(dist, d)
        far = jnp.argmax(dist).astype(_i32)
        return dist, far, cent

    init = (jnp.full((N,), 1e10, _f32), jnp.int32(0), jnp.zeros((S,), _i32))
    _, _, cent = lax.fori_loop(0, S, body, init)
    out_ref[0, 0, :] = cent


def _run_fps(combined):
    return pl.pallas_call(
        _fps_body,
        grid=(B,),
        in_specs=[pl.BlockSpec((1, N, C0), lambda b: (b, 0, 0))],
        out_specs=pl.BlockSpec((1, 1, S), lambda b: (b, 0, 0)),
        out_shape=jax.ShapeDtypeStruct((B, 1, S), _i32),
    )(combined)


def _fps_body_t(t_ref, out_ref):
    xt = t_ref[...]
    iota_row = lax.broadcasted_iota(_i32, (1, N), 1)
    lane_row = lax.broadcasted_iota(_i32, (1, S), 1)

    def body(i, state):
        dist, idxs, cents = state
        cents = [jnp.where(lane_row == i, idxs[b], cents[b])
                 for b in range(B)]
        masks = [(iota_row == idxs[b]).astype(_f32) for b in range(B)]
        cs = [jax.lax.dot_general(
                  xt[b], masks[b], (((1,), (1,)), ((), ())),
                  preferred_element_type=_f32)[None]
              for b in range(B)]
        c = jnp.concatenate(cs, axis=0)
        sq = xt - c
        d = jnp.sum(sq * sq, axis=1)
        dist = jnp.minimum(dist, d)
        idxs = []
        for b in range(B):
            row = dist[b]
            m = jnp.max(row)
            idxs.append(jnp.min(jnp.where(row == m, iota_row[0],
                                          jnp.int32(N))))
        return dist, idxs, cents

    init = (jnp.full((B, N), 1e10, _f32), [jnp.int32(0)] * B,
            [jnp.zeros((1, S), _i32)] * B)
    _, _, cents = lax.fori_loop(0, S, body, init)
    out_ref[...] = jnp.concatenate(cents, axis=0)[:, None, :]


def _run_fps_t(combined_t):
    return pl.pallas_call(
        _fps_body_t,
        grid=(1,),
        in_specs=[pl.BlockSpec((B, C0, N), lambda i: (0, 0, 0))],
        out_specs=pl.BlockSpec((B, 1, S), lambda i: (0, 0, 0)),
        out_shape=jax.ShapeDtypeStruct((B, 1, S), _i32),
    )(combined_t)


def _knn_body(c_ref, fps_ref, q_ref, idx_ref):
    cb = c_ref[0]
    fps = fps_ref[0, 0, :]
    onehot = (fps[:, None] ==
              lax.broadcasted_iota(_i32, (S, N), 1)).astype(_f32)
    q = jax.lax.dot_general(onehot, cb, (((1,), (0,)), ((), ())),
                            preferred_element_type=_f32)
    q_ref[0] = q
    qf, qx = q[:, 3:23], q[:, 0:3]
    cf, cx = cb[:, 3:23], cb[:, 0:3]
    d1 = -2.0 * jax.lax.dot_general(qf, cf, (((1,), (1,)), ((), ())),
                                    preferred_element_type=_f32)
    d1 = d1 + jnp.sum(qf * qf, axis=1)[:, None]
    d1 = d1 + jnp.sum(cf * cf, axis=1)[None, :]
    d2 = -2.0 * jax.lax.dot_general(qx, cx, (((1,), (1,)), ((), ())),
                                    preferred_element_type=_f32)
    d2 = d2 + jnp.sum(qx * qx, axis=1)[:, None]
    d2 = d2 + jnp.sum(cx * cx, axis=1)[None, :]
    dist = d1 + d2
    off = pl.program_id(0) * N
    lane_n = lax.broadcasted_iota(_i32, (S, N), 1)
    cols = []
    for _ in range(K):
        m = jnp.argmin(dist, axis=1).astype(_i32)
        cols.append(m + off)
        dist = jnp.where(lane_n == m[:, None], 1e30, dist)
    idx_ref[0] = jnp.stack(cols, axis=1)


def _run_knn(combined, fps):
    return pl.pallas_call(
        _knn_body,
        grid=(B,),
        in_specs=[
            pl.BlockSpec((1, N, C0), lambda b: (b, 0, 0)),
            pl.BlockSpec((1, 1, S), lambda b: (b, 0, 0)),
        ],
        out_specs=[
            pl.BlockSpec((1, S, C0), lambda b: (b, 0, 0)),
            pl.BlockSpec((1, S, K), lambda b: (b, 0, 0)),
        ],
        out_shape=[
            jax.ShapeDtypeStruct((B, S, C0), _f32),
            jax.ShapeDtypeStruct((B, S, K), _i32),
        ],
    )(combined, fps)


def _sc_gather(table, idx):
    info = plsc.get_sparse_core_info()
    nw = info.num_cores * info.num_subcores
    b_per_w = ROWS // nw
    chunk = 512
    nchunks = b_per_w // chunk
    mesh = plsc.VectorSubcoreMesh(core_axis_name="c", subcore_axis_name="s")

    @functools.partial(
        pl.kernel,
        mesh=mesh,
        out_type=jax.ShapeDtypeStruct((ROWS, CT), _f32),
        scratch_types=[
            pltpu.VMEM((chunk,), _i32),
            pltpu.VMEM((chunk, CT), _f32),
            pltpu.SemaphoreType.DMA,
        ],
    )
    def gather_k(table_hbm, idx_hbm, out_hbm, idx_v, rows_v, sem):
        wid = lax.axis_index("s") * info.num_cores + lax.axis_index("c")
        base = wid * b_per_w

        def body(ci, carry):
            off = base + ci * chunk
            pltpu.sync_copy(idx_hbm.at[pl.ds(off, chunk)], idx_v)
            pltpu.async_copy(table_hbm.at[idx_v], rows_v, sem).wait()
            pltpu.sync_copy(rows_v, out_hbm.at[pl.ds(off, chunk)])
            return carry

        lax.fori_loop(0, nchunks, body, 0)

    return gather_k(table, idx)


def _l0_body(g_ref, qc_ref, wg_ref, wq_ref, p_ref, y_ref, s0_ref, s2_ref):
    g = g_ref[...]
    qc = qc_ref[...]
    y = jax.lax.dot_general(g, wg_ref[...], (((1,), (1,)), ((), ())),
                            preferred_element_type=_f32)
    y = y + jax.lax.dot_general(qc, wq_ref[...], (((1,), (1,)), ((), ())),
                                preferred_element_type=_f32)
    gx = g[:, 0:3] - qc[:, 0:3]
    sq = jnp.sum(gx * gx, axis=1)
    nrm = jnp.where(sq > 0, jnp.sqrt(jnp.where(sq > 0, sq, 1.0)), 0.0)
    y = y + nrm[:, None] * p_ref[0:1, :] + p_ref[1:2, :]
    y_ref[...] = y

    @pl.when(pl.program_id(0) == 0)
    def _():
        s0_ref[...] = jnp.zeros_like(s0_ref)
        s2_ref[...] = jnp.zeros_like(s2_ref)

    s0_ref[...] += jnp.sum(y, axis=0)[None, None, :]
    s2_ref[...] += jnp.sum(y * y, axis=0)[None, None, :]


def _run_l0(g, qc, wg, wq, p0):
    return pl.pallas_call(
        _l0_body,
        grid=(NSTEP,),
        in_specs=[
            pl.BlockSpec((RT, CT), lambda i: (i, 0)),
            pl.BlockSpec((RT, C0), lambda i: (i, 0)),
            pl.BlockSpec((64, CT), lambda i: (0, 0)),
            pl.BlockSpec((64, C0), lambda i: (0, 0)),
            pl.BlockSpec((8, 64), lambda i: (0, 0)),
        ],
        out_specs=[
            pl.BlockSpec((RT, 64), lambda i: (i, 0)),
            pl.BlockSpec((1, 1, 64), lambda i: (0, 0, 0)),
            pl.BlockSpec((1, 1, 64), lambda i: (0, 0, 0)),
        ],
        out_shape=[
            jax.ShapeDtypeStruct((ROWS, 64), _f32),
            jax.ShapeDtypeStruct((1, 1, 64), _f32),
            jax.ShapeDtypeStruct((1, 1, 64), _f32),
        ],
    )(g, qc, wg, wq, p0)


def _mid_body(y_ref, w_ref, p_ref, o_ref, s0_ref, s2_ref):
    z = jnp.maximum(y_ref[...] * p_ref[0:1, 0:64] + p_ref[1:2, 0:64], 0.0)
    y = jax.lax.dot_general(z, w_ref[...], (((1,), (1,)), ((), ())),
                            preferred_element_type=_f32)
    y = y + p_ref[2:3, 0:w_ref.shape[0]]
    o_ref[...] = y

    @pl.when(pl.program_id(0) == 0)
    def _():
        s0_ref[...] = jnp.zeros_like(s0_ref)
        s2_ref[...] = jnp.zeros_like(s2_ref)

    s0_ref[...] += jnp.sum(y, axis=0)[None, None, :]
    s2_ref[...] += jnp.sum(y * y, axis=0)[None, None, :]


def _run_mid(y, w, p, out_ch):
    return pl.pallas_call(
        _mid_body,
        grid=(NSTEP,),
        in_specs=[
            pl.BlockSpec((RT, 64), lambda i: (i, 0)),
            pl.BlockSpec((out_ch, 64), lambda i: (0, 0)),
            pl.BlockSpec((8, 128), lambda i: (0, 0)),
        ],
        out_specs=[
            pl.BlockSpec((RT, out_ch), lambda i: (i, 0)),
            pl.BlockSpec((1, 1, out_ch), lambda i: (0, 0, 0)),
            pl.BlockSpec((1, 1, out_ch), lambda i: (0, 0, 0)),
        ],
        out_shape=[
            jax.ShapeDtypeStruct((ROWS, out_ch), _f32),
            jax.ShapeDtypeStruct((1, 1, out_ch), _f32),
            jax.ShapeDtypeStruct((1, 1, out_ch), _f32),
        ],
    )(y, w, p)


def _last_body(y_ref, w_ref, p_ref, mx_ref, mn_ref, s0_ref, s2_ref):
    z = jnp.maximum(y_ref[...] * p_ref[0:1, 0:64] + p_ref[1:2, 0:64], 0.0)
    y = jax.lax.dot_general(z, w_ref[...], (((1,), (1,)), ((), ())),
                            preferred_element_type=_f32)
    y = y + p_ref[2:3, :]
    y3 = y.reshape(RT // K, K, 128)
    mx_ref[...] = jnp.max(y3, axis=1)
    mn_ref[...] = jnp.min(y3, axis=1)

    @pl.when(pl.program_id(0) == 0)
    def _():
        s0_ref[...] = jnp.zeros_like(s0_ref)
        s2_ref[...] = jnp.zeros_like(s2_ref)

    s0_ref[...] += jnp.sum(y, axis=0)[None, None, :]
    s2_ref[...] += jnp.sum(y * y, axis=0)[None, None, :]


def _run_last(y, w, p):
    return pl.pallas_call(
        _last_body,
        grid=(NSTEP,),
        in_specs=[
            pl.BlockSpec((RT, 64), lambda i: (i, 0)),
            pl.BlockSpec((128, 64), lambda i: (0, 0)),
            pl.BlockSpec((8, 128), lambda i: (0, 0)),
        ],
        out_specs=[
            pl.BlockSpec((RT // K, 128), lambda i: (i, 0)),
            pl.BlockSpec((RT // K, 128), lambda i: (i, 0)),
            pl.BlockSpec((1, 1, 128), lambda i: (0, 0, 0)),
            pl.BlockSpec((1, 1, 128), lambda i: (0, 0, 0)),
        ],
        out_shape=[
            jax.ShapeDtypeStruct((B * S, 128), _f32),
            jax.ShapeDtypeStruct((B * S, 128), _f32),
            jax.ShapeDtypeStruct((1, 1, 128), _f32),
            jax.ShapeDtypeStruct((1, 1, 128), _f32),
        ],
    )(y, w, p)


def _fin_body(mx_ref, mn_ref, p_ref, o_ref):
    a = p_ref[0:1, :]
    c = p_ref[1:2, :]
    pick = jnp.where(a > 0, mx_ref[...], mn_ref[...])
    o_ref[...] = jnp.maximum(pick * a + c, 0.0)


def _run_fin(mx, mn, p):
    return pl.pallas_call(
        _fin_body,
        grid=(1,),
        in_specs=[
            pl.BlockSpec((B * S, 128), lambda i: (0, 0)),
            pl.BlockSpec((B * S, 128), lambda i: (0, 0)),
            pl.BlockSpec((8, 128), lambda i: (0, 0)),
        ],
        out_specs=pl.BlockSpec((B * S, 128), lambda i: (0, 0)),
        out_shape=jax.ShapeDtypeStruct((B * S, 128), _f32),
    )(mx, mn, p)


def _bn_affine(s0, s2, g, beta):
    mean = s0.reshape(-1) / ROWS
    var = s2.reshape(-1) / ROWS - mean * mean
    a = g / jnp.sqrt(var + 1e-5)
    return a, beta - mean * a


def kernel(xyz, feature, points, W0, b0, g0, beta0, W1, b1, g1, beta1,
           W2, b2, g2, beta2):
    xyz_t = jnp.transpose(xyz, (0, 2, 1))
    pts_t = jnp.transpose(points, (0, 2, 1))
    combined = jnp.concatenate(
        [xyz_t, feature, jnp.zeros((B, N, 1), _f32)], axis=-1)

    fps = _run_fps_t(jnp.transpose(combined, (0, 2, 1)))
    qfull, gidx = _run_knn(combined, fps)

    table = jnp.concatenate(
        [xyz_t, feature, pts_t, jnp.zeros((B, N, CT - 87), _f32)],
        axis=-1).reshape(B * N, CT)
    g = _sc_gather(table, gidx.reshape(ROWS))

    qc = jnp.repeat(qfull.reshape(B * S, C0), K, axis=0)

    wg = jnp.concatenate(
        [W0[:, 0:3], W0[:, 4:24], W0[:, 24:88],
         jnp.zeros((64, CT - 87), _f32)], axis=1)
    wq = jnp.concatenate(
        [-W0[:, 0:3], -W0[:, 4:24], jnp.zeros((64, 1), _f32)],
        axis=1)
    p0 = jnp.zeros((8, 64), _f32).at[0].set(W0[:, 3]).at[1].set(b0)

    y0, s0a, s0b = _run_l0(g, qc, wg, wq, p0)
    a0, c0 = _bn_affine(s0a, s0b, g0, beta0)

    p1 = (jnp.zeros((8, 128), _f32)
          .at[0, 0:64].set(a0).at[1, 0:64].set(c0).at[2, 0:64].set(b1))
    y1, s1a, s1b = _run_mid(y0, W1, p1, 64)
    a1, c1 = _bn_affine(s1a, s1b, g1, beta1)

    p2 = (jnp.zeros((8, 128), _f32)
          .at[0, 0:64].set(a1).at[1, 0:64].set(c1).at[2, 0:128].set(b2))
    mx, mn, s2a, s2b = _run_last(y1, W2, p2)
    a2, c2 = _bn_affine(s2a, s2b, g2, beta2)

    p3 = jnp.zeros((8, 128), _f32).at[0].set(a2).at[1].set(c2)
    pooled = _run_fin(mx, mn, p3)

    new_xyz = jnp.transpose(qfull[:, :, 0:3], (0, 2, 1))
    new_bin_map = qfull[:, :, 3:23]
    new_points_out = jnp.transpose(pooled.reshape(B, S, 128), (0, 2, 1))
    return new_xyz, new_points_out, new_bin_map


# speedup vs baseline: 11.2528x; 1.4347x over previous
import functools

import jax
import jax.numpy as jnp
from jax import lax
from jax.experimental import pallas as pl
from jax.experimental.pallas import tpu as pltpu
from jax.experimental.pallas import tpu_sc as plsc

B, N, S, K = 8, 2048, 512, 32
D_F, D_P = 20, 64
C0 = 24
CT = 128
ROWS = B * S * K
RT = 4096
NSTEP = ROWS // RT

_f32 = jnp.float32
_i32 = jnp.int32


def _fps_body(c_ref, out_ref):
    cb = c_ref[0]
    lane_s = lax.broadcasted_iota(_i32, (S,), 0)

    def body(i, state):
        dist, far, cent = state
        cent = jnp.where(lane_s == i, far, cent)
        c = c_ref[0, pl.ds(far, 1), :]
        d = jnp.sum((cb - c) ** 2, axis=1)
        dist = jnp.minimum(dist, d)
        far = jnp.argmax(dist).astype(_i32)
        return dist, far, cent

    init = (jnp.full((N,), 1e10, _f32), jnp.int32(0), jnp.zeros((S,), _i32))
    _, _, cent = lax.fori_loop(0, S, body, init)
    out_ref[0, 0, :] = cent


def _run_fps(combined):
    return pl.pallas_call(
        _fps_body,
        grid=(B,),
        in_specs=[pl.BlockSpec((1, N, C0), lambda b: (b, 0, 0))],
        out_specs=pl.BlockSpec((1, 1, S), lambda b: (b, 0, 0)),
        out_shape=jax.ShapeDtypeStruct((B, 1, S), _i32),
    )(combined)


def _fps_body_t(t_ref, out_ref):
    xt = t_ref[...]
    iota2 = lax.broadcasted_iota(_i32, (B, N), 1)
    lane2 = lax.broadcasted_iota(_i32, (B, S), 1)
    mask0 = (iota2 == 0).astype(_f32)

    def body(i, state):
        dist, mask, cent = state
        cs = [jax.lax.dot_general(
                  xt[b], mask[b:b + 1], (((1,), (1,)), ((), ())),
                  preferred_element_type=_f32)[None]
              for b in range(B)]
        c = jnp.concatenate(cs, axis=0)
        sq = xt - c
        d = jnp.sum(sq * sq, axis=1)
        dist = jnp.minimum(dist, d)
        m2 = jnp.max(dist, axis=1, keepdims=True)
        cand = jnp.where(dist == m2, iota2, jnp.int32(N))
        idxv = jnp.min(cand, axis=1, keepdims=True)
        cent = jnp.where(lane2 == i + 1, idxv, cent)
        mask = (iota2 == idxv).astype(_f32)
        return dist, mask, cent

    init = (jnp.full((B, N), 1e10, _f32), mask0, jnp.zeros((B, S), _i32))
    _, _, cent = lax.fori_loop(0, S, body, init)
    out_ref[...] = cent[:, None, :]


def _run_fps_t(combined_t):
    return pl.pallas_call(
        _fps_body_t,
        grid=(1,),
        in_specs=[pl.BlockSpec((B, C0, N), lambda i: (0, 0, 0))],
        out_specs=pl.BlockSpec((B, 1, S), lambda i: (0, 0, 0)),
        out_shape=jax.ShapeDtypeStruct((B, 1, S), _i32),
    )(combined_t)


def _knn_body(c_ref, fps_ref, q_ref, idx_ref):
    cb = c_ref[0]
    fps = fps_ref[0, 0, :]
    onehot = (fps[:, None] ==
              lax.broadcasted_iota(_i32, (S, N), 1)).astype(_f32)
    q = jax.lax.dot_general(onehot, cb, (((1,), (0,)), ((), ())),
                            preferred_element_type=_f32)
    q_ref[0] = q
    qf, qx = q[:, 3:23], q[:, 0:3]
    cf, cx = cb[:, 3:23], cb[:, 0:3]
    d1 = -2.0 * jax.lax.dot_general(qf, cf, (((1,), (1,)), ((), ())),
                                    preferred_element_type=_f32)
    d1 = d1 + jnp.sum(qf * qf, axis=1)[:, None]
    d1 = d1 + jnp.sum(cf * cf, axis=1)[None, :]
    d2 = -2.0 * jax.lax.dot_general(qx, cx, (((1,), (1,)), ((), ())),
                                    preferred_element_type=_f32)
    d2 = d2 + jnp.sum(qx * qx, axis=1)[:, None]
    d2 = d2 + jnp.sum(cx * cx, axis=1)[None, :]
    dist = d1 + d2
    off = pl.program_id(0) * N
    lane_n = lax.broadcasted_iota(_i32, (S, N), 1)
    cols = []
    for _ in range(K):
        m = jnp.argmin(dist, axis=1).astype(_i32)
        cols.append(m + off)
        dist = jnp.where(lane_n == m[:, None], 1e30, dist)
    idx_ref[0] = jnp.stack(cols, axis=1)


def _run_knn(combined, fps):
    return pl.pallas_call(
        _knn_body,
        grid=(B,),
        in_specs=[
            pl.BlockSpec((1, N, C0), lambda b: (b, 0, 0)),
            pl.BlockSpec((1, 1, S), lambda b: (b, 0, 0)),
        ],
        out_specs=[
            pl.BlockSpec((1, S, C0), lambda b: (b, 0, 0)),
            pl.BlockSpec((1, S, K), lambda b: (b, 0, 0)),
        ],
        out_shape=[
            jax.ShapeDtypeStruct((B, S, C0), _f32),
            jax.ShapeDtypeStruct((B, S, K), _i32),
        ],
    )(combined, fps)


def _sc_gather(table, idx):
    info = plsc.get_sparse_core_info()
    nw = info.num_cores * info.num_subcores
    b_per_w = ROWS // nw
    chunk = 512
    nchunks = b_per_w // chunk
    mesh = plsc.VectorSubcoreMesh(core_axis_name="c", subcore_axis_name="s")

    @functools.partial(
        pl.kernel,
        mesh=mesh,
        out_type=jax.ShapeDtypeStruct((ROWS, CT), _f32),
        scratch_types=[
            pltpu.VMEM((chunk,), _i32),
            pltpu.VMEM((chunk, CT), _f32),
            pltpu.SemaphoreType.DMA,
        ],
    )
    def gather_k(table_hbm, idx_hbm, out_hbm, idx_v, rows_v, sem):
        wid = lax.axis_index("s") * info.num_cores + lax.axis_index("c")
        base = wid * b_per_w

        def body(ci, carry):
            off = base + ci * chunk
            pltpu.sync_copy(idx_hbm.at[pl.ds(off, chunk)], idx_v)
            pltpu.async_copy(table_hbm.at[idx_v], rows_v, sem).wait()
            pltpu.sync_copy(rows_v, out_hbm.at[pl.ds(off, chunk)])
            return carry

        lax.fori_loop(0, nchunks, body, 0)

    return gather_k(table, idx)


def _l0_body(g_ref, qc_ref, wg_ref, wq_ref, p_ref, y_ref, s0_ref, s2_ref):
    g = g_ref[...]
    qc = qc_ref[...]
    y = jax.lax.dot_general(g, wg_ref[...], (((1,), (1,)), ((), ())),
                            preferred_element_type=_f32)
    y = y + jax.lax.dot_general(qc, wq_ref[...], (((1,), (1,)), ((), ())),
                                preferred_element_type=_f32)
    gx = g[:, 0:3] - qc[:, 0:3]
    sq = jnp.sum(gx * gx, axis=1)
    nrm = jnp.where(sq > 0, jnp.sqrt(jnp.where(sq > 0, sq, 1.0)), 0.0)
    y = y + nrm[:, None] * p_ref[0:1, :] + p_ref[1:2, :]
    y_ref[...] = y

    @pl.when(pl.program_id(0) == 0)
    def _():
        s0_ref[...] = jnp.zeros_like(s0_ref)
        s2_ref[...] = jnp.zeros_like(s2_ref)

    s0_ref[...] += jnp.sum(y, axis=0)[None, None, :]
    s2_ref[...] += jnp.sum(y * y, axis=0)[None, None, :]


def _run_l0(g, qc, wg, wq, p0):
    return pl.pallas_call(
        _l0_body,
        grid=(NSTEP,),
        in_specs=[
            pl.BlockSpec((RT, CT), lambda i: (i, 0)),
            pl.BlockSpec((RT, C0), lambda i: (i, 0)),
            pl.BlockSpec((64, CT), lambda i: (0, 0)),
            pl.BlockSpec((64, C0), lambda i: (0, 0)),
            pl.BlockSpec((8, 64), lambda i: (0, 0)),
        ],
        out_specs=[
            pl.BlockSpec((RT, 64), lambda i: (i, 0)),
            pl.BlockSpec((1, 1, 64), lambda i: (0, 0, 0)),
            pl.BlockSpec((1, 1, 64), lambda i: (0, 0, 0)),
        ],
        out_shape=[
            jax.ShapeDtypeStruct((ROWS, 64), _f32),
            jax.ShapeDtypeStruct((1, 1, 64), _f32),
            jax.ShapeDtypeStruct((1, 1, 64), _f32),
        ],
    )(g, qc, wg, wq, p0)


def _mid_body(y_ref, w_ref, p_ref, o_ref, s0_ref, s2_ref):
    z = jnp.maximum(y_ref[...] * p_ref[0:1, 0:64] + p_ref[1:2, 0:64], 0.0)
    y = jax.lax.dot_general(z, w_ref[...], (((1,), (1,)), ((), ())),
                            preferred_element_type=_f32)
    y = y + p_ref[2:3, 0:w_ref.shape[0]]
    o_ref[...] = y

    @pl.when(pl.program_id(0) == 0)
    def _():
        s0_ref[...] = jnp.zeros_like(s0_ref)
        s2_ref[...] = jnp.zeros_like(s2_ref)

    s0_ref[...] += jnp.sum(y, axis=0)[None, None, :]
    s2_ref[...] += jnp.sum(y * y, axis=0)[None, None, :]


def _run_mid(y, w, p, out_ch):
    return pl.pallas_call(
        _mid_body,
        grid=(NSTEP,),
        in_specs=[
            pl.BlockSpec((RT, 64), lambda i: (i, 0)),
            pl.BlockSpec((out_ch, 64), lambda i: (0, 0)),
            pl.BlockSpec((8, 128), lambda i: (0, 0)),
        ],
        out_specs=[
            pl.BlockSpec((RT, out_ch), lambda i: (i, 0)),
            pl.BlockSpec((1, 1, out_ch), lambda i: (0, 0, 0)),
            pl.BlockSpec((1, 1, out_ch), lambda i: (0, 0, 0)),
        ],
        out_shape=[
            jax.ShapeDtypeStruct((ROWS, out_ch), _f32),
            jax.ShapeDtypeStruct((1, 1, out_ch), _f32),
            jax.ShapeDtypeStruct((1, 1, out_ch), _f32),
        ],
    )(y, w, p)


def _last_body(y_ref, w_ref, p_ref, mx_ref, mn_ref, s0_ref, s2_ref):
    z = jnp.maximum(y_ref[...] * p_ref[0:1, 0:64] + p_ref[1:2, 0:64], 0.0)
    y = jax.lax.dot_general(z, w_ref[...], (((1,), (1,)), ((), ())),
                            preferred_element_type=_f32)
    y = y + p_ref[2:3, :]
    y3 = y.reshape(RT // K, K, 128)
    mx_ref[...] = jnp.max(y3, axis=1)
    mn_ref[...] = jnp.min(y3, axis=1)

    @pl.when(pl.program_id(0) == 0)
    def _():
        s0_ref[...] = jnp.zeros_like(s0_ref)
        s2_ref[...] = jnp.zeros_like(s2_ref)

    s0_ref[...] += jnp.sum(y, axis=0)[None, None, :]
    s2_ref[...] += jnp.sum(y * y, axis=0)[None, None, :]


def _run_last(y, w, p):
    return pl.pallas_call(
        _last_body,
        grid=(NSTEP,),
        in_specs=[
            pl.BlockSpec((RT, 64), lambda i: (i, 0)),
            pl.BlockSpec((128, 64), lambda i: (0, 0)),
            pl.BlockSpec((8, 128), lambda i: (0, 0)),
        ],
        out_specs=[
            pl.BlockSpec((RT // K, 128), lambda i: (i, 0)),
            pl.BlockSpec((RT // K, 128), lambda i: (i, 0)),
            pl.BlockSpec((1, 1, 128), lambda i: (0, 0, 0)),
            pl.BlockSpec((1, 1, 128), lambda i: (0, 0, 0)),
        ],
        out_shape=[
            jax.ShapeDtypeStruct((B * S, 128), _f32),
            jax.ShapeDtypeStruct((B * S, 128), _f32),
            jax.ShapeDtypeStruct((1, 1, 128), _f32),
            jax.ShapeDtypeStruct((1, 1, 128), _f32),
        ],
    )(y, w, p)


def _fin_body(mx_ref, mn_ref, p_ref, o_ref):
    a = p_ref[0:1, :]
    c = p_ref[1:2, :]
    pick = jnp.where(a > 0, mx_ref[...], mn_ref[...])
    o_ref[...] = jnp.maximum(pick * a + c, 0.0)


def _run_fin(mx, mn, p):
    return pl.pallas_call(
        _fin_body,
        grid=(1,),
        in_specs=[
            pl.BlockSpec((B * S, 128), lambda i: (0, 0)),
            pl.BlockSpec((B * S, 128), lambda i: (0, 0)),
            pl.BlockSpec((8, 128), lambda i: (0, 0)),
        ],
        out_specs=pl.BlockSpec((B * S, 128), lambda i: (0, 0)),
        out_shape=jax.ShapeDtypeStruct((B * S, 128), _f32),
    )(mx, mn, p)


def _bn_affine(s0, s2, g, beta):
    mean = s0.reshape(-1) / ROWS
    var = s2.reshape(-1) / ROWS - mean * mean
    a = g / jnp.sqrt(var + 1e-5)
    return a, beta - mean * a


def kernel(xyz, feature, points, W0, b0, g0, beta0, W1, b1, g1, beta1,
           W2, b2, g2, beta2):
    xyz_t = jnp.transpose(xyz, (0, 2, 1))
    pts_t = jnp.transpose(points, (0, 2, 1))
    combined = jnp.concatenate(
        [xyz_t, feature, jnp.zeros((B, N, 1), _f32)], axis=-1)

    fps = _run_fps_t(jnp.transpose(combined, (0, 2, 1)))
    qfull, gidx = _run_knn(combined, fps)

    table = jnp.concatenate(
        [xyz_t, feature, pts_t, jnp.zeros((B, N, CT - 87), _f32)],
        axis=-1).reshape(B * N, CT)
    g = _sc_gather(table, gidx.reshape(ROWS))

    qc = jnp.repeat(qfull.reshape(B * S, C0), K, axis=0)

    wg = jnp.concatenate(
        [W0[:, 0:3], W0[:, 4:24], W0[:, 24:88],
         jnp.zeros((64, CT - 87), _f32)], axis=1)
    wq = jnp.concatenate(
        [-W0[:, 0:3], -W0[:, 4:24], jnp.zeros((64, 1), _f32)],
        axis=1)
    p0 = jnp.zeros((8, 64), _f32).at[0].set(W0[:, 3]).at[1].set(b0)

    y0, s0a, s0b = _run_l0(g, qc, wg, wq, p0)
    a0, c0 = _bn_affine(s0a, s0b, g0, beta0)

    p1 = (jnp.zeros((8, 128), _f32)
          .at[0, 0:64].set(a0).at[1, 0:64].set(c0).at[2, 0:64].set(b1))
    y1, s1a, s1b = _run_mid(y0, W1, p1, 64)
    a1, c1 = _bn_affine(s1a, s1b, g1, beta1)

    p2 = (jnp.zeros((8, 128), _f32)
          .at[0, 0:64].set(a1).at[1, 0:64].set(c1).at[2, 0:128].set(b2))
    mx, mn, s2a, s2b = _run_last(y1, W2, p2)
    a2, c2 = _bn_affine(s2a, s2b, g2, beta2)

    p3 = jnp.zeros((8, 128), _f32).at[0].set(a2).at[1].set(c2)
    pooled = _run_fin(mx, mn, p3)

    new_xyz = jnp.transpose(qfull[:, :, 0:3], (0, 2, 1))
    new_bin_map = qfull[:, :, 3:23]
    new_points_out = jnp.transpose(pooled.reshape(B, S, 128), (0, 2, 1))
    return new_xyz, new_points_out, new_bin_map
